# baseline (speedup 1.0000x reference)
import jax
import jax.numpy as jnp
from jax.experimental import pallas as pl
from jax.experimental.pallas import tpu as pltpu

_GAMMA = 3.0
_BELT = 1.2
_LAMN = 10.0


def _train_gather(tables, sides, sidx, T, RT=8):
    n_tbl = len(tables)

    def tg_index(side, c):
        def im(i, sref):
            return (sref[side * T + i * RT + c], 0, 0)
        return im

    ins = []
    in_specs = []
    for tbl, sd in zip(tables, sides):
        N, D = tbl.shape
        t3 = tbl.reshape(N, 1, D)
        for c in range(RT):
            ins.append(t3)
            in_specs.append(pl.BlockSpec((1, 1, D), tg_index(sd, c)))
    out_specs = [
        pl.BlockSpec((RT, tbl.shape[1]), lambda i, sref: (i, 0))
        for tbl in tables
    ]
    out_shape = [
        jax.ShapeDtypeStruct((T, tbl.shape[1]), jnp.float32) for tbl in tables
    ]

    def body(sref, *refs):
        in_refs = refs[: n_tbl * RT]
        out_refs = refs[n_tbl * RT:]
        for t in range(n_tbl):
            for c in range(RT):
                out_refs[t][c, :] = in_refs[t * RT + c][0, 0, :]

    return pl.pallas_call(
        body,
        grid_spec=pltpu.PrefetchScalarGridSpec(
            num_scalar_prefetch=1,
            grid=(T // RT,),
            in_specs=in_specs,
            out_specs=out_specs,
        ),
        out_shape=out_shape,
    )(sidx, *ins)


def _dist_call(table, tb2, ttr, K, T, R=32):
    N, D = table.shape
    nst = K * (T // R)
    g = tb2.reshape(2, K, T // R, R).transpose(0, 2, 1, 3).reshape(-1)
    t3 = table.reshape(N, 1, D)

    def neg_im(c):
        return lambda i, sref: (sref[i * R + c], 0, 0)

    in_specs = [pl.BlockSpec((1, 1, D), neg_im(c)) for c in range(R)]
    in_specs.append(
        pl.BlockSpec((1, R, D), lambda i, sref: (i // nst, (i % nst) // K, 0))
    )
    out_spec = pl.BlockSpec((1, R, 1), lambda i, sref: (i, 0, 0))

    def body(sref, *refs):
        negs = refs[:R]
        ttr_ref = refs[R]
        out_ref = refs[R + 1]
        rows = jnp.concatenate([r[0] for r in negs], axis=0)
        out_ref[0] = jnp.sum(
            jnp.abs(rows - ttr_ref[0]), axis=-1, keepdims=True
        )

    return pl.pallas_call(
        body,
        grid_spec=pltpu.PrefetchScalarGridSpec(
            num_scalar_prefetch=1,
            grid=(2 * nst,),
            in_specs=in_specs,
            out_specs=out_spec,
        ),
        out_shape=jax.ShapeDtypeStruct((2 * nst, R, 1), jnp.float32),
    )(g, *([t3] * R), ttr)


def _reduce(dX, dN, dO, tx0, tx1, tn0, tn1, to0, to1, denom):
    def body(dx_ref, dn_ref, do_ref, x0, x1r, n0, n1, o0, o1, out_ref):
        acc = jnp.float32(0.0)
        groups = (
            (dx_ref, x0, x1r, _GAMMA),
            (dn_ref, n0, n1, _BELT),
            (do_ref, o0, o1, _BELT),
        )
        for d_ref, a_ref, b_ref, margin in groups:
            mS = jnp.sum(
                jnp.abs(a_ref[...] - b_ref[...]), axis=-1, keepdims=True
            )
            m3 = mS.reshape(8, 64, 1)
            d = d_ref[...][None]

            for t in range(8):
                mc3 = m3[t].reshape(64, 1, 1)
                v = jnp.maximum(margin + (mc3 - d) * (1.0 / _LAMN), 0.0)
                acc = acc + jnp.sum(v)
        out_ref[...] = jnp.full((1, 1), acc * denom, jnp.float32)

    return pl.pallas_call(
        body,
        out_shape=jax.ShapeDtypeStruct((1, 1), jnp.float32),
    )(dX, dN, dO, tx0, tx1, tn0, tn1, to0, to1)


def kernel(x1, x2, x_name1, x_name2, onehot1, onehot2, train_set, train_batch):
    T = train_set.shape[0]
    K = train_batch.shape[1]
    sidx = jnp.concatenate([train_set[:, 0], train_set[:, 1]])

    tx0, tx1, tn0, tn1, to0, to1 = _train_gather(
        [x1, x2, x_name1, x_name2, onehot1, onehot2],
        [0, 1, 0, 1, 0, 1],
        sidx,
        T,
    )

    ttrX = jnp.stack([tx0, tx1])
    ttrN = jnp.stack([tn0, tn1])
    ttrO = jnp.stack([to0, to1])
    tb = train_batch

    dx_a = _dist_call(x1, jnp.stack([tb[0], tb[3]]), ttrX, K, T)
    dx_b = _dist_call(x2, jnp.stack([tb[1], tb[2]]), ttrX, K, T)
    dn_a = _dist_call(x_name1, jnp.stack([tb[4], tb[7]]), ttrN, K, T)
    dn_b = _dist_call(x_name2, jnp.stack([tb[5], tb[6]]), ttrN, K, T)
    do_a = _dist_call(onehot1, jnp.stack([tb[8], tb[11]]), ttrO, K, T)
    do_b = _dist_call(onehot2, jnp.stack([tb[9], tb[10]]), ttrO, K, T)

    dX = jnp.concatenate([dx_a.reshape(-1), dx_b.reshape(-1)]).reshape(256, 128)
    dN = jnp.concatenate([dn_a.reshape(-1), dn_b.reshape(-1)]).reshape(256, 128)
    dO = jnp.concatenate([do_a.reshape(-1), do_b.reshape(-1)]).reshape(256, 128)

    denom = 1.0 / (4.0 * K * T * T)
    out = _reduce(dX, dN, dO, tx0, tx1, tn0, tn1, to0, to1, denom)
    return out[0, 0]
